# Initial kernel scaffold; baseline (speedup 1.0000x reference)
import jax
import jax.numpy as jnp
from jax.experimental import pallas as pl


def kernel(x, lengths, lengths_per_path, n_paths_per_batch, W_ih, W_hh, b_ih, b_hh, W_fc, b_fc):
    raise NotImplementedError("write your pallas kernel here")


# speedup vs baseline: 3.8775x; 3.8775x over previous
import jax
import jax.numpy as jnp
from jax.experimental import pallas as pl
from jax.experimental.pallas import tpu as pltpu

B, T, F, H, O = 8, 2048, 128, 128, 128
G = 4 * H
CHUNK = 256
NCHUNK = T // CHUNK

_HIGH = jax.lax.Precision.HIGHEST


def _lstm_kernel(lens_ref, x_ref, wih_ref, whh_ref, wfc_ref, bg_ref, bfc_ref,
                 out_ref, hT_ref, cT_ref,
                 xg_ref, ho_ref, h_ref, c_ref):
    pid = pl.program_id(0)
    start = pid * CHUNK
    lens = lens_ref[...]
    nsteps = jnp.max(lens)
    steps = jnp.clip(nsteps - start, 0, CHUNK)

    @pl.when(pid == 0)
    def _init():
        h_ref[...] = jnp.zeros_like(h_ref)
        c_ref[...] = jnp.zeros_like(c_ref)

    @pl.when(steps > 0)
    def _work():
        xb = x_ref[...]
        xg = jax.lax.dot_general(xb, wih_ref[...], (((2,), (0,)), ((), ())),
                                 precision=_HIGH,
                                 preferred_element_type=jnp.float32)
        xg_ref[...] = xg + bg_ref[...][None]

        ho_ref[...] = jnp.zeros_like(ho_ref)

        def body(t, carry):
            h, c = carry
            gates = xg_ref[t] + jnp.dot(h, whh_ref[...], precision=_HIGH,
                                        preferred_element_type=jnp.float32)
            i_g = jax.nn.sigmoid(gates[:, 0:H])
            f_g = jax.nn.sigmoid(gates[:, H:2 * H])
            g_g = jnp.tanh(gates[:, 2 * H:3 * H])
            o_g = jax.nn.sigmoid(gates[:, 3 * H:4 * H])
            c_new = f_g * c + i_g * g_g
            h_new = o_g * jnp.tanh(c_new)
            m = (start + t) < lens
            ho_ref[t] = jnp.where(m, h_new, 0.0)
            return jnp.where(m, h_new, h), jnp.where(m, c_new, c)

        h1, c1 = jax.lax.fori_loop(0, steps, body, (h_ref[...], c_ref[...]))
        h_ref[...] = h1
        c_ref[...] = c1

        fc = jax.lax.dot_general(ho_ref[...], wfc_ref[...],
                                 (((2,), (0,)), ((), ())),
                                 precision=_HIGH,
                                 preferred_element_type=jnp.float32)
        fc = fc + bfc_ref[...][None]
        tio = jax.lax.broadcasted_iota(jnp.int32, (CHUNK, B, O), 0) + start
        out_ref[...] = jnp.where(tio < lens[None], fc, 0.0)

    @pl.when(steps <= 0)
    def _skip():
        out_ref[...] = jnp.zeros_like(out_ref)

    hT_ref[...] = h_ref[...]
    cT_ref[...] = c_ref[...]


def kernel(x, lengths, lengths_per_path, n_paths_per_batch,
           W_ih, W_hh, b_ih, b_hh, W_fc, b_fc):
    del lengths
    lpp = jnp.asarray(lengths_per_path)
    npb = jnp.asarray(n_paths_per_batch)
    n_paths = lpp.shape[1]
    valid = jnp.arange(n_paths)[None, :] < npb[:, None]
    lens = jnp.where(valid, lpp, 0)[:, 0].astype(jnp.int32)
    lens_v = jnp.broadcast_to(lens[:, None], (B, H)).astype(jnp.int32)

    x_t = jnp.transpose(x, (1, 0, 2))
    bg = jnp.broadcast_to((b_ih + b_hh)[None, :], (B, G))
    bfc = jnp.broadcast_to(b_fc[None, :], (B, O))

    full = lambda *s: pl.BlockSpec(s, lambda i: (0,) * len(s))
    out_t, hT, cT = pl.pallas_call(
        _lstm_kernel,
        grid=(NCHUNK,),
        in_specs=[
            full(B, H),
            pl.BlockSpec((CHUNK, B, F), lambda i: (i, 0, 0)),
            full(F, G),
            full(H, G),
            full(H, O),
            full(B, G),
            full(B, O),
        ],
        out_specs=[
            pl.BlockSpec((CHUNK, B, O), lambda i: (i, 0, 0)),
            full(B, H),
            full(B, H),
        ],
        out_shape=[
            jax.ShapeDtypeStruct((T, B, O), jnp.float32),
            jax.ShapeDtypeStruct((B, H), jnp.float32),
            jax.ShapeDtypeStruct((B, H), jnp.float32),
        ],
        scratch_shapes=[
            pltpu.VMEM((CHUNK, B, G), jnp.float32),
            pltpu.VMEM((CHUNK, B, H), jnp.float32),
            pltpu.VMEM((B, H), jnp.float32),
            pltpu.VMEM((B, H), jnp.float32),
        ],
        compiler_params=pltpu.CompilerParams(
            dimension_semantics=("arbitrary",),
        ),
    )(lens_v, x_t, W_ih.T, W_hh.T, W_fc.T, bg, bfc)

    out_final = jnp.transpose(out_t, (1, 0, 2))
    return (out_final, hT[None], cT[None])


# speedup vs baseline: 6.5064x; 1.6780x over previous
import jax
import jax.numpy as jnp
from jax.experimental import pallas as pl
from jax.experimental.pallas import tpu as pltpu

B, T, F, H, O = 8, 2048, 128, 128, 128
G = 4 * H
CHUNK = 256
NCHUNK = T // CHUNK

_PREC = jax.lax.Precision.DEFAULT


def _lstm_kernel(lens_ref, x_ref, wih_ref, whh_ref, wfc_ref, bg_ref, bfc_ref,
                 out_ref, hT_ref, cT_ref,
                 xg_ref, ho_ref, h_ref, c_ref):
    pid = pl.program_id(0)
    start = pid * CHUNK
    lens = lens_ref[...]
    nsteps = jnp.max(lens)
    steps = jnp.clip(nsteps - start, 0, CHUNK)

    @pl.when(pid == 0)
    def _init():
        h_ref[...] = jnp.zeros_like(h_ref)
        c_ref[...] = jnp.zeros_like(c_ref)

    @pl.when(steps > 0)
    def _work():
        xb = x_ref[...]
        xg = jax.lax.dot_general(xb, wih_ref[...], (((2,), (0,)), ((), ())),
                                 precision=_PREC,
                                 preferred_element_type=jnp.float32)
        xg_ref[...] = xg + bg_ref[...][None]

        ho_ref[...] = jnp.zeros_like(ho_ref)

        def body(t, carry):
            h, c = carry
            gates = xg_ref[t] + jnp.dot(h.astype(jnp.bfloat16),
                                        whh_ref[...].astype(jnp.bfloat16),
                                        preferred_element_type=jnp.float32)
            i_g = jax.nn.sigmoid(gates[:, 0:H])
            f_g = jax.nn.sigmoid(gates[:, H:2 * H])
            g_g = jnp.tanh(gates[:, 2 * H:3 * H])
            o_g = jax.nn.sigmoid(gates[:, 3 * H:4 * H])
            c_new = f_g * c + i_g * g_g
            h_new = o_g * jnp.tanh(c_new)
            m = (start + t) < lens
            ho_ref[t] = jnp.where(m, h_new, 0.0)
            return jnp.where(m, h_new, h), jnp.where(m, c_new, c)

        h1, c1 = jax.lax.fori_loop(0, steps, body, (h_ref[...], c_ref[...]))
        h_ref[...] = h1
        c_ref[...] = c1

        fc = jax.lax.dot_general(ho_ref[...], wfc_ref[...],
                                 (((2,), (0,)), ((), ())),
                                 precision=_PREC,
                                 preferred_element_type=jnp.float32)
        fc = fc + bfc_ref[...][None]
        tio = jax.lax.broadcasted_iota(jnp.int32, (CHUNK, B, O), 0) + start
        out_ref[...] = jnp.where(tio < lens[None], fc, 0.0)

    @pl.when(steps <= 0)
    def _skip():
        out_ref[...] = jnp.zeros_like(out_ref)

    hT_ref[...] = h_ref[...]
    cT_ref[...] = c_ref[...]


def kernel(x, lengths, lengths_per_path, n_paths_per_batch,
           W_ih, W_hh, b_ih, b_hh, W_fc, b_fc):
    del lengths
    lpp = jnp.asarray(lengths_per_path)
    npb = jnp.asarray(n_paths_per_batch)
    n_paths = lpp.shape[1]
    valid = jnp.arange(n_paths)[None, :] < npb[:, None]
    lens = jnp.where(valid, lpp, 0)[:, 0].astype(jnp.int32)
    lens_v = jnp.broadcast_to(lens[:, None], (B, H)).astype(jnp.int32)

    x_t = jnp.transpose(x, (1, 0, 2))
    bg = jnp.broadcast_to((b_ih + b_hh)[None, :], (B, G))
    bfc = jnp.broadcast_to(b_fc[None, :], (B, O))

    full = lambda *s: pl.BlockSpec(s, lambda i: (0,) * len(s))
    out_t, hT, cT = pl.pallas_call(
        _lstm_kernel,
        grid=(NCHUNK,),
        in_specs=[
            full(B, H),
            pl.BlockSpec((CHUNK, B, F), lambda i: (i, 0, 0)),
            full(F, G),
            full(H, G),
            full(H, O),
            full(B, G),
            full(B, O),
        ],
        out_specs=[
            pl.BlockSpec((CHUNK, B, O), lambda i: (i, 0, 0)),
            full(B, H),
            full(B, H),
        ],
        out_shape=[
            jax.ShapeDtypeStruct((T, B, O), jnp.float32),
            jax.ShapeDtypeStruct((B, H), jnp.float32),
            jax.ShapeDtypeStruct((B, H), jnp.float32),
        ],
        scratch_shapes=[
            pltpu.VMEM((CHUNK, B, G), jnp.float32),
            pltpu.VMEM((CHUNK, B, H), jnp.float32),
            pltpu.VMEM((B, H), jnp.float32),
            pltpu.VMEM((B, H), jnp.float32),
        ],
        compiler_params=pltpu.CompilerParams(
            dimension_semantics=("arbitrary",),
        ),
    )(lens_v, x_t, W_ih.T, W_hh.T, W_fc.T, bg, bfc)

    out_final = jnp.transpose(out_t, (1, 0, 2))
    return (out_final, hT[None], cT[None])
